# Initial kernel scaffold; baseline (speedup 1.0000x reference)
import functools

import jax
import jax.numpy as jnp
from jax import lax
from jax.experimental import pallas as pl
from jax.experimental.pallas import tpu as pltpu
from jax.experimental.pallas import tpu_sc as plsc

N = 10000
E = 160000
D = 256
H = 8
DH = 32
HALF = 128
WROW = 144
NTILES = 16
EPT = E // NTILES
CH = 80
NCHUNK = EPT // CH
NROWS_PT = N // NTILES
ZCH = 125
NZ = NROWS_PT // ZCH
INV_SQRT_DH = 1.0 / (DH ** 0.5)

_mesh = plsc.VectorSubcoreMesh(core_axis_name="c", subcore_axis_name="s")


@functools.partial(
    pl.kernel,
    mesh=_mesh,
    out_type=jax.ShapeDtypeStruct((2, N, WROW), jnp.float32),
    scratch_types=[
        pltpu.VMEM((CH,), jnp.int32),
        pltpu.VMEM((CH,), jnp.int32),
        pltpu.VMEM((CH,), jnp.int32),
        pltpu.VMEM((CH, HALF), jnp.float32),
        pltpu.VMEM((CH, HALF), jnp.float32),
        pltpu.VMEM((CH, HALF), jnp.float32),
        pltpu.VMEM((CH, WROW), jnp.float32),
        pltpu.VMEM((ZCH, WROW), jnp.float32),
        pltpu.VMEM_SHARED((N, WROW), jnp.float32),
        pltpu.SemaphoreType.DMA,
        pltpu.SemaphoreType.DMA,
        pltpu.SemaphoreType.DMA,
    ],
)
def _edge_kernel(src_hbm, dst_hbm, qcat, kcat, vcat, out_hbm,
                 isrc, idst, idstl, qr, kr, vr, wr, zbuf, acc,
                 sem_q, sem_k, sem_v):
    c = lax.axis_index("c")
    s = lax.axis_index("s")

    zeros16 = jnp.zeros((16,), jnp.float32)

    def zrow(i, carry):
        for j in range(WROW // 16):
            zbuf[i, pl.ds(j * 16, 16)] = zeros16
        return carry

    lax.fori_loop(0, ZCH, zrow, 0)
    for z in range(NZ):
        pltpu.sync_copy(zbuf, acc.at[pl.ds(s * NROWS_PT + z * ZCH, ZCH)])
    plsc.subcore_barrier()

    coff = c * N
    lane = lax.broadcasted_iota(jnp.int32, (16,), 0)

    def chunk_body(g, carry):
        ebase = s * EPT + g * CH
        pltpu.sync_copy(src_hbm.at[pl.ds(ebase, CH)], isrc)
        pltpu.sync_copy(dst_hbm.at[pl.ds(ebase, CH)], idst)
        for j in range(CH // 16):
            sl = pl.ds(j * 16, 16)
            sv = isrc[sl]
            dv = idst[sl]
            idstl[sl] = dv
            isrc[sl] = sv + coff
            idst[sl] = dv + coff
        cp_q = pltpu.async_copy(qcat.at[isrc], qr, sem_q)
        cp_k = pltpu.async_copy(kcat.at[idst], kr, sem_k)
        cp_v = pltpu.async_copy(vcat.at[isrc], vr, sem_v)
        cp_q.wait()
        cp_k.wait()
        cp_v.wait()

        def edge_body(i, ecarry):
            prods = []
            for j in range(8):
                sl = pl.ds(j * 16, 16)
                prods.append(qr[i, sl] * kr[i, sl])
            exvecs = []
            for h in range(4):
                s2 = prods[2 * h] + prods[2 * h + 1]
                tot = jnp.sum(s2) * INV_SQRT_DH
                exvecs.append(jnp.exp(jnp.full((16,), tot, jnp.float32)))
            for j in range(8):
                sl = pl.ds(j * 16, 16)
                wr[i, sl] = vr[i, sl] * exvecs[j // 2]
            evec = jnp.zeros((16,), jnp.float32)
            for h in range(4):
                evec = jnp.where(lane == h, exvecs[h], evec)
            wr[i, pl.ds(HALF, 16)] = evec
            return ecarry

        lax.fori_loop(0, CH, edge_body, 0)
        pltpu.sync_copy(wr, acc.at[idstl], add=True)
        return carry

    lax.fori_loop(0, NCHUNK, chunk_body, 0)

    plsc.subcore_barrier()
    for z in range(NZ):
        r0 = s * NROWS_PT + z * ZCH
        pltpu.sync_copy(acc.at[pl.ds(r0, ZCH)], out_hbm.at[c, pl.ds(r0, ZCH)])


def _proj_body(x_ref, w_ref, b_ref, o_ref):
    o_ref[...] = jnp.dot(x_ref[...], w_ref[...],
                         preferred_element_type=jnp.float32) + b_ref[...]


def _final_body(o_ref, d_ref, w_ref, b_ref, z_ref):
    d = jnp.maximum(d_ref[...], 1e-30)
    z_ref[...] = jnp.dot(o_ref[...] / d, w_ref[...],
                         preferred_element_type=jnp.float32) + b_ref[...]


_MB = 400


def _proj(x, wcat, bcat):
    return pl.pallas_call(
        _proj_body,
        grid=(N // _MB,),
        in_specs=[
            pl.BlockSpec((_MB, D), lambda i: (i, 0)),
            pl.BlockSpec((D, 3 * D), lambda i: (0, 0)),
            pl.BlockSpec((1, 3 * D), lambda i: (0, 0)),
        ],
        out_specs=pl.BlockSpec((_MB, 3 * D), lambda i: (i, 0)),
        out_shape=jax.ShapeDtypeStruct((N, 3 * D), jnp.float32),
    )(x, wcat, bcat)


def _final(o, dfull, wo, bo):
    return pl.pallas_call(
        _final_body,
        grid=(N // _MB,),
        in_specs=[
            pl.BlockSpec((_MB, D), lambda i: (i, 0)),
            pl.BlockSpec((_MB, D), lambda i: (i, 0)),
            pl.BlockSpec((D, D), lambda i: (0, 0)),
            pl.BlockSpec((1, D), lambda i: (0, 0)),
        ],
        out_specs=pl.BlockSpec((_MB, D), lambda i: (i, 0)),
        out_shape=jax.ShapeDtypeStruct((N, D), jnp.float32),
    )(o, dfull, wo, bo)


def _halves(a):
    return a.reshape(N, 2, HALF).transpose(1, 0, 2).reshape(2 * N, HALF)


def kernel(x, edge_index, Wq, bq, Wk, bk, Wv, bv, Wo, bo):
    src = edge_index[0].astype(jnp.int32)
    dst = edge_index[1].astype(jnp.int32)
    wcat = jnp.concatenate([Wq, Wk, Wv], axis=1)
    bcat = jnp.concatenate([bq, bk, bv]).reshape(1, 3 * D)
    y = _proj(x, wcat, bcat)
    q = y[:, :D]
    k = y[:, D:2 * D]
    v = y[:, 2 * D:]
    out = _edge_kernel(src, dst, _halves(q), _halves(k), _halves(v))
    o_un = jnp.concatenate([out[0, :, :HALF], out[1, :, :HALF]], axis=1)
    den = jnp.concatenate([out[0, :, HALF:HALF + 4],
                           out[1, :, HALF:HALF + 4]], axis=1)
    dfull = jnp.repeat(den, DH, axis=1)
    return _final(o_un, dfull, Wo, bo.reshape(1, D))


# speedup vs baseline: 23.0121x; 23.0121x over previous
import functools

import jax
import jax.numpy as jnp
from jax import lax
from jax.experimental import pallas as pl
from jax.experimental.pallas import tpu as pltpu
from jax.experimental.pallas import tpu_sc as plsc

N = 10000
E = 160000
D = 256
H = 8
DH = 32
HALF = 128
NTILES = 16
EPT = E // NTILES
CH = 80
NCHUNK = EPT // CH
NP = 10240
NROWS_PT = NP // NTILES
NZ = NROWS_PT // CH
NB2 = NP // 8
B2PT = NB2 // NTILES
INV_SQRT_DH = 1.0 / (DH ** 0.5)

_mesh = plsc.VectorSubcoreMesh(core_axis_name="c", subcore_axis_name="s")


@functools.partial(
    pl.kernel,
    mesh=_mesh,
    out_type=(
        jax.ShapeDtypeStruct((2, NP, HALF), jnp.float32),
        jax.ShapeDtypeStruct((2, NB2, HALF), jnp.float32),
    ),
    scratch_types=[
        pltpu.VMEM((CH,), jnp.int32),
        pltpu.VMEM((CH,), jnp.int32),
        pltpu.VMEM((CH,), jnp.int32),
        pltpu.VMEM((CH,), jnp.int32),
        pltpu.VMEM((CH, HALF), jnp.float32),
        pltpu.VMEM((CH, HALF), jnp.float32),
        pltpu.VMEM((CH, HALF), jnp.float32),
        pltpu.VMEM_SHARED((NP, HALF), jnp.float32),
        pltpu.VMEM_SHARED((NB2, HALF), jnp.float32),
        pltpu.SemaphoreType.DMA,
        pltpu.SemaphoreType.DMA,
        pltpu.SemaphoreType.DMA,
    ],
)
def _edge_kernel(src_hbm, dst_hbm, qcat, kcat, vcat, out_hbm, den_hbm,
                 isrc, idst, idstl, idx2, qr, kr, vr, acc, acc2,
                 sem_q, sem_k, sem_v):
    c = lax.axis_index("c")
    s = lax.axis_index("s")

    zeros16 = jnp.zeros((16,), jnp.float32)

    def zrow(i, carry):
        for j in range(HALF // 16):
            qr[i, pl.ds(j * 16, 16)] = zeros16
        return carry

    lax.fori_loop(0, CH, zrow, 0)
    for z in range(NZ):
        pltpu.sync_copy(qr, acc.at[pl.ds(s * NROWS_PT + z * CH, CH)])
    pltpu.sync_copy(qr, acc2.at[pl.ds(s * B2PT, B2PT)])
    plsc.subcore_barrier()

    coff = c * N
    lane = lax.broadcasted_iota(jnp.int32, (16,), 0)
    perms = [lane ^ k for k in (1, 2, 4, 8)]
    _dnums = lax.GatherDimensionNumbers(
        offset_dims=(), collapsed_slice_dims=(0,), start_index_map=(0,))

    def _vtake(vv, idx):
        return lax.gather(vv, idx[:, None], dimension_numbers=_dnums,
                          slice_sizes=(1,),
                          mode=lax.GatherScatterMode.PROMISE_IN_BOUNDS)

    def allsum(vv):
        for p in perms:
            vv = vv + _vtake(vv, p)
        return vv

    def chunk_body(g, carry):
        ebase = s * EPT + g * CH
        pltpu.sync_copy(src_hbm.at[pl.ds(ebase, CH)], isrc)
        pltpu.sync_copy(dst_hbm.at[pl.ds(ebase, CH)], idst)
        for j in range(CH // 16):
            sl = pl.ds(j * 16, 16)
            sv = isrc[sl]
            dv = idst[sl]
            idstl[sl] = dv
            idx2[sl] = lax.shift_right_logical(dv, 3)
            isrc[sl] = sv + coff
            idst[sl] = dv + coff
        cp_q = pltpu.async_copy(qcat.at[isrc], qr, sem_q)
        cp_k = pltpu.async_copy(kcat.at[idst], kr, sem_k)
        cp_v = pltpu.async_copy(vcat.at[isrc], vr, sem_v)
        cp_q.wait()
        cp_k.wait()
        cp_v.wait()

        def group_body(g2, ecarry):
            dvec16 = idstl[pl.ds(g2 * 16, 16)]
            for e in range(16):
                i = g2 * 16 + e
                prods = []
                for j in range(8):
                    sl = pl.ds(j * 16, 16)
                    prods.append(qr[i, sl] * kr[i, sl])
                exvecs = []
                for h in range(4):
                    s2 = prods[2 * h] + prods[2 * h + 1]
                    exvecs.append(jnp.exp(allsum(s2) * INV_SQRT_DH))
                for j in range(8):
                    sl = pl.ds(j * 16, 16)
                    qr[i, sl] = vr[i, sl] * exvecs[j // 2]
                evec = jnp.zeros((16,), jnp.float32)
                for h in range(4):
                    evec = jnp.where(lane == h, exvecs[h], evec)
                for j in range(8):
                    kr[i, pl.ds(j * 16, 16)] = zeros16
                off = pl.multiple_of((dvec16[e] & 7) * 16, 16)
                kr[i, pl.ds(off, 16)] = evec
            return ecarry

        lax.fori_loop(0, CH // 16, group_body, 0)
        pltpu.sync_copy(qr, acc.at[idstl], add=True)
        pltpu.sync_copy(kr, acc2.at[idx2], add=True)
        return carry

    lax.fori_loop(0, NCHUNK, chunk_body, 0)

    plsc.subcore_barrier()
    for z in range(NZ):
        r0 = s * NROWS_PT + z * CH
        pltpu.sync_copy(acc.at[pl.ds(r0, CH)], out_hbm.at[c, pl.ds(r0, CH)])
    b0 = s * B2PT
    pltpu.sync_copy(acc2.at[pl.ds(b0, B2PT)], den_hbm.at[c, pl.ds(b0, B2PT)])


def _proj_body(x_ref, w_ref, b_ref, o_ref):
    o_ref[...] = jnp.dot(x_ref[...], w_ref[...],
                         preferred_element_type=jnp.float32) + b_ref[...]


def _final_body(o_ref, d_ref, s_ref, w_ref, b_ref, z_ref):
    dfull = jnp.dot(d_ref[...], s_ref[...], preferred_element_type=jnp.float32)
    d = jnp.maximum(dfull, 1e-30)
    z_ref[...] = jnp.dot(o_ref[...] / d, w_ref[...],
                         preferred_element_type=jnp.float32) + b_ref[...]


_MB = 400


def _proj(x, wcat, bcat):
    return pl.pallas_call(
        _proj_body,
        grid=(N // _MB,),
        in_specs=[
            pl.BlockSpec((_MB, D), lambda i: (i, 0)),
            pl.BlockSpec((D, 3 * D), lambda i: (0, 0)),
            pl.BlockSpec((1, 3 * D), lambda i: (0, 0)),
        ],
        out_specs=pl.BlockSpec((_MB, 3 * D), lambda i: (i, 0)),
        out_shape=jax.ShapeDtypeStruct((N, 3 * D), jnp.float32),
    )(x, wcat, bcat)


def _final(o, dflat, sel, wo, bo):
    return pl.pallas_call(
        _final_body,
        grid=(N // _MB,),
        in_specs=[
            pl.BlockSpec((_MB, D), lambda i: (i, 0)),
            pl.BlockSpec((_MB, 32), lambda i: (i, 0)),
            pl.BlockSpec((32, D), lambda i: (0, 0)),
            pl.BlockSpec((D, D), lambda i: (0, 0)),
            pl.BlockSpec((1, D), lambda i: (0, 0)),
        ],
        out_specs=pl.BlockSpec((_MB, D), lambda i: (i, 0)),
        out_shape=jax.ShapeDtypeStruct((N, D), jnp.float32),
    )(o, dflat, sel, wo, bo)


def _halves(a):
    return a.reshape(N, 2, HALF).transpose(1, 0, 2).reshape(2 * N, HALF)


def kernel(x, edge_index, Wq, bq, Wk, bk, Wv, bv, Wo, bo):
    src = edge_index[0].astype(jnp.int32)
    dst = edge_index[1].astype(jnp.int32)
    wcat = jnp.concatenate([Wq, Wk, Wv], axis=1)
    bcat = jnp.concatenate([bq, bk, bv]).reshape(1, 3 * D)
    y = _proj(x, wcat, bcat)
    q = y[:, :D]
    k = y[:, D:2 * D]
    v = y[:, 2 * D:]
    out, dbuckets = _edge_kernel(src, dst, _halves(q), _halves(k), _halves(v))
    o_un = jnp.concatenate([out[0, :N, :], out[1, :N, :]], axis=1)
    dflat = dbuckets.reshape(2, NB2, 8, 16).transpose(1, 2, 0, 3)
    dflat = dflat.reshape(NP, 32)[:N]
    r = jnp.arange(32)
    head = (r // 16) * 4 + (r % 16)
    valid = (r % 16) < 4
    sel = ((jnp.arange(D)[None, :] // DH == head[:, None]) &
           valid[:, None]).astype(jnp.float32)
    return _final(o_un, dflat, sel, Wo, bo.reshape(1, D))
